# Initial kernel scaffold; baseline (speedup 1.0000x reference)
import jax
import jax.numpy as jnp
from jax.experimental import pallas as pl


def kernel(indices, table):
    raise NotImplementedError("write your pallas kernel here")


# speedup vs baseline: 1.5766x; 1.5766x over previous
import functools

import jax
import jax.numpy as jnp
from jax import lax
from jax.experimental import pallas as pl
from jax.experimental.pallas import tpu as pltpu
from jax.experimental.pallas import tpu_sc as plsc

_NUM_EMB = 1000000
_D = 32
_B = 16384
_F = 26
_TOT = _B * _F

_NC = 2
_NS = 16
_NW = _NC * _NS
_B_PER_W = _TOT // _NW
_CHUNK = 1664
_NCH = _B_PER_W // _CHUNK

_mesh = plsc.VectorSubcoreMesh(core_axis_name="c", subcore_axis_name="s")


@functools.partial(
    pl.kernel,
    mesh=_mesh,
    compiler_params=pltpu.CompilerParams(use_tc_tiling_on_sc=False),
    out_type=jax.ShapeDtypeStruct((_TOT, _D), jnp.float32),
    scratch_types=[
        pltpu.VMEM((_NCH, _CHUNK), jnp.int32),
        pltpu.VMEM((2, _CHUNK, _D), jnp.float32),
        pltpu.SemaphoreType.DMA,
        pltpu.SemaphoreType.DMA,
        pltpu.SemaphoreType.DMA,
        pltpu.SemaphoreType.DMA,
    ],
)
def _emb_gather(table_hbm, idx_hbm, out_hbm, idx_v, rows_v, g0, g1, w0, w1):
    wid = lax.axis_index("s") * _NC + lax.axis_index("c")
    base = wid * _B_PER_W
    pltpu.sync_copy(idx_hbm.at[pl.ds(wid * _NCH, _NCH)], idx_v)

    gsems = (g0, g1)
    wsems = (w0, w1)
    gathers = [None, None]
    writes = [None, None]
    for b in range(2):
        gathers[b] = pltpu.async_copy(
            table_hbm.at[idx_v.at[b]], rows_v.at[b], gsems[b])
    for g in range(_NCH):
        b = g & 1
        gathers[b].wait()
        writes[b] = pltpu.async_copy(
            rows_v.at[b], out_hbm.at[pl.ds(base + g * _CHUNK, _CHUNK)],
            wsems[b])
        if g + 2 < _NCH:
            writes[b].wait()
            gathers[b] = pltpu.async_copy(
                table_hbm.at[idx_v.at[g + 2]], rows_v.at[b], gsems[b])
    writes[(_NCH - 2) & 1].wait()
    writes[(_NCH - 1) & 1].wait()


def kernel(indices, table):
    idx = indices.astype(jnp.int32).reshape(_NW * _NCH, _CHUNK)
    out = _emb_gather(table, idx)
    return out.reshape(_B, _F, _D)


# speedup vs baseline: 1.5770x; 1.0002x over previous
import functools

import jax
import jax.numpy as jnp
from jax import lax
from jax.experimental import pallas as pl
from jax.experimental.pallas import tpu as pltpu
from jax.experimental.pallas import tpu_sc as plsc

_NUM_EMB = 1000000
_D = 32
_B = 16384
_F = 26
_TOT = _B * _F

_NC = 2
_NS = 16
_NW = _NC * _NS
_B_PER_W = _TOT // _NW
_CHUNK = 1664
_NCH = _B_PER_W // _CHUNK

_mesh = plsc.VectorSubcoreMesh(core_axis_name="c", subcore_axis_name="s")


@functools.partial(
    pl.kernel,
    mesh=_mesh,
    compiler_params=pltpu.CompilerParams(use_tc_tiling_on_sc=False),
    out_type=jax.ShapeDtypeStruct((_TOT, _D), jnp.float32),
    scratch_types=[
        pltpu.VMEM((_B_PER_W,), jnp.int32),
        pltpu.VMEM((2, _CHUNK, _D), jnp.float32),
        pltpu.SemaphoreType.DMA,
        pltpu.SemaphoreType.DMA,
        pltpu.SemaphoreType.DMA,
        pltpu.SemaphoreType.DMA,
    ],
)
def _emb_gather(table_hbm, idx_hbm, out_hbm, idx_v, rows_v, g0, g1, w0, w1):
    wid = lax.axis_index("s") * _NC + lax.axis_index("c")
    base = wid * _B_PER_W
    pltpu.sync_copy(idx_hbm.at[pl.ds(base, _B_PER_W)], idx_v)

    gsems = (g0, g1)
    wsems = (w0, w1)
    gathers = [None, None]
    writes = [None, None]
    for b in range(2):
        gathers[b] = pltpu.async_copy(
            table_hbm.at[idx_v.at[pl.ds(b * _CHUNK, _CHUNK)]],
            rows_v.at[b], gsems[b])
    for g in range(_NCH):
        b = g & 1
        gathers[b].wait()
        writes[b] = pltpu.async_copy(
            rows_v.at[b], out_hbm.at[pl.ds(base + g * _CHUNK, _CHUNK)],
            wsems[b])
        if g + 2 < _NCH:
            writes[b].wait()
            gathers[b] = pltpu.async_copy(
                table_hbm.at[idx_v.at[pl.ds((g + 2) * _CHUNK, _CHUNK)]],
                rows_v.at[b], gsems[b])
    writes[(_NCH - 2) & 1].wait()
    writes[(_NCH - 1) & 1].wait()


def kernel(indices, table):
    idx = indices.astype(jnp.int32).reshape(_TOT)
    out = _emb_gather(table, idx)
    return out.reshape(_B, _F, _D)
